# Initial kernel scaffold; baseline (speedup 1.0000x reference)
import functools

import jax
import jax.numpy as jnp
from jax import lax
from jax.experimental import pallas as pl
from jax.experimental.pallas import tpu as pltpu
from jax.experimental.pallas import tpu_sc as plsc

_BATCH = 16384

_info = plsc.get_sparse_core_info()
_NC = _info.num_cores
_NS = _info.num_subcores
_NW = _NC * _NS
_B_PER_W = _BATCH // _NW
_CHUNK = 128
_NCHUNK = _B_PER_W // _CHUNK
_LANES = 16


@functools.partial(
    pl.kernel,
    mesh=plsc.VectorSubcoreMesh(core_axis_name="c", subcore_axis_name="s"),
    out_type=jax.ShapeDtypeStruct((_BATCH,), jnp.float32),
    scratch_types=[
        pltpu.VMEM((_NCHUNK, _CHUNK), jnp.int32),
        pltpu.VMEM((_NCHUNK, _CHUNK), jnp.int32),
        pltpu.VMEM((_NCHUNK, _CHUNK), jnp.float32),
        pltpu.VMEM((_NCHUNK, _CHUNK), jnp.float32),
        pltpu.SemaphoreType.DMA,
        pltpu.SemaphoreType.DMA,
    ],
)
def _irt_sc(aidx_hbm, tidx_hbm, th_hbm, be_hbm, out_hbm,
            aidx_v, tidx_v, th_v, be_v, sem_a, sem_b):
    wid = lax.axis_index("s") * _NC + lax.axis_index("c")
    base = wid * _B_PER_W

    pltpu.sync_copy(
        aidx_hbm.at[pl.ds(base, _B_PER_W)],
        aidx_v.reshape(_B_PER_W),
    )
    pltpu.sync_copy(
        tidx_hbm.at[pl.ds(base, _B_PER_W)],
        tidx_v.reshape(_B_PER_W),
    )

    copies = []
    for j in range(_NCHUNK):
        copies.append(pltpu.async_copy(th_hbm.at[aidx_v.at[j]], th_v.at[j], sem_a))
        copies.append(pltpu.async_copy(be_hbm.at[tidx_v.at[j]], be_v.at[j], sem_b))
    for cp in copies:
        cp.wait()

    for j in range(_NCHUNK):
        for i in range(_CHUNK // _LANES):
            sl = pl.ds(i * _LANES, _LANES)
            th_v.at[j][sl] = th_v.at[j][sl] - be_v.at[j][sl]

    pltpu.sync_copy(
        th_v.reshape(_B_PER_W),
        out_hbm.at[pl.ds(base, _B_PER_W)],
    )


def kernel(agent_idx, task_idx, theta, beta):
    agent_idx = agent_idx.astype(jnp.int32)
    task_idx = task_idx.astype(jnp.int32)
    theta_flat = theta.reshape(-1)
    beta_flat = beta.reshape(-1)
    return _irt_sc(agent_idx, task_idx, theta_flat, beta_flat)


# speedup vs baseline: 1.0746x; 1.0746x over previous
import functools

import jax
import jax.numpy as jnp
from jax import lax
from jax.experimental import pallas as pl
from jax.experimental.pallas import tpu as pltpu
from jax.experimental.pallas import tpu_sc as plsc

_BATCH = 16384

_info = plsc.get_sparse_core_info()
_NC = _info.num_cores
_NS = _info.num_subcores
_NW = _NC * _NS
_B_PER_W = _BATCH // _NW
_CHUNK = 128
_NCHUNK = _B_PER_W // _CHUNK
_LANES = 16


@functools.partial(
    pl.kernel,
    mesh=plsc.VectorSubcoreMesh(core_axis_name="c", subcore_axis_name="s"),
    out_type=jax.ShapeDtypeStruct((_BATCH,), jnp.float32),
    scratch_types=[
        pltpu.VMEM((_NCHUNK, _CHUNK), jnp.int32),
        pltpu.VMEM((_NCHUNK, _CHUNK), jnp.int32),
        pltpu.VMEM((_NCHUNK, _CHUNK), jnp.float32),
        pltpu.VMEM((_NCHUNK, _CHUNK), jnp.float32),
        pltpu.SemaphoreType.DMA,
        pltpu.SemaphoreType.DMA,
    ],
)
def _irt_sc(aidx_hbm, tidx_hbm, th_hbm, be_hbm, out_hbm,
            aidx_v, tidx_v, th_v, be_v, sem_a, sem_b):
    wid = lax.axis_index("s") * _NC + lax.axis_index("c")
    base = wid * _B_PER_W

    for j in range(_NCHUNK):
        pltpu.sync_copy(
            aidx_hbm.at[pl.ds(base + j * _CHUNK, _CHUNK)],
            aidx_v.at[j],
        )
        pltpu.sync_copy(
            tidx_hbm.at[pl.ds(base + j * _CHUNK, _CHUNK)],
            tidx_v.at[j],
        )

    copies = []
    for j in range(_NCHUNK):
        copies.append(pltpu.async_copy(th_hbm.at[aidx_v.at[j]], th_v.at[j], sem_a))
        copies.append(pltpu.async_copy(be_hbm.at[tidx_v.at[j]], be_v.at[j], sem_b))
    for cp in copies:
        cp.wait()

    for j in range(_NCHUNK):
        for i in range(_CHUNK // _LANES):
            sl = pl.ds(i * _LANES, _LANES)
            th_v.at[j][sl] = th_v.at[j][sl] - be_v.at[j][sl]

    for j in range(_NCHUNK):
        pltpu.sync_copy(
            th_v.at[j],
            out_hbm.at[pl.ds(base + j * _CHUNK, _CHUNK)],
        )


def kernel(agent_idx, task_idx, theta, beta):
    agent_idx = agent_idx.astype(jnp.int32)
    task_idx = task_idx.astype(jnp.int32)
    theta_flat = theta.reshape(-1)
    beta_flat = beta.reshape(-1)
    return _irt_sc(agent_idx, task_idx, theta_flat, beta_flat)


# speedup vs baseline: 1.1235x; 1.0454x over previous
import functools

import jax
import jax.numpy as jnp
from jax import lax
from jax.experimental import pallas as pl
from jax.experimental.pallas import tpu as pltpu
from jax.experimental.pallas import tpu_sc as plsc

_BATCH = 16384

_info = plsc.get_sparse_core_info()
_NC = _info.num_cores
_NS = _info.num_subcores
_NW = _NC * _NS
_B_PER_W = _BATCH // _NW
_CHUNK = 128
_NCHUNK = _B_PER_W // _CHUNK
_LANES = 16


@functools.partial(
    pl.kernel,
    mesh=plsc.VectorSubcoreMesh(core_axis_name="c", subcore_axis_name="s"),
    out_type=jax.ShapeDtypeStruct((_NW, _NCHUNK, _CHUNK), jnp.float32),
    scratch_types=[
        pltpu.VMEM((_NCHUNK, _CHUNK), jnp.int32),
        pltpu.VMEM((_NCHUNK, _CHUNK), jnp.int32),
        pltpu.VMEM((_NCHUNK, _CHUNK), jnp.float32),
        pltpu.VMEM((_NCHUNK, _CHUNK), jnp.float32),
        pltpu.SemaphoreType.DMA,
        pltpu.SemaphoreType.DMA,
        pltpu.SemaphoreType.DMA,
    ],
)
def _irt_sc(aidx_hbm, tidx_hbm, th_hbm, be_hbm, out_hbm,
            aidx_v, tidx_v, th_v, be_v, sem_ai, sem_ti, sem_g):
    wid = lax.axis_index("s") * _NC + lax.axis_index("c")

    cp_a = pltpu.async_copy(aidx_hbm.at[wid], aidx_v, sem_ai)
    cp_t = pltpu.async_copy(tidx_hbm.at[wid], tidx_v, sem_ti)

    gathers = []
    cp_a.wait()
    for j in range(_NCHUNK):
        gathers.append(
            pltpu.async_copy(th_hbm.at[aidx_v.at[j]], th_v.at[j], sem_g))
    cp_t.wait()
    for j in range(_NCHUNK):
        gathers.append(
            pltpu.async_copy(be_hbm.at[tidx_v.at[j]], be_v.at[j], sem_g))
    for cp in gathers:
        cp.wait()

    for j in range(_NCHUNK):
        for i in range(_CHUNK // _LANES):
            sl = pl.ds(i * _LANES, _LANES)
            th_v.at[j][sl] = th_v.at[j][sl] - be_v.at[j][sl]

    pltpu.sync_copy(th_v, out_hbm.at[wid])


def kernel(agent_idx, task_idx, theta, beta):
    agent_idx = agent_idx.astype(jnp.int32).reshape(_NW, _NCHUNK, _CHUNK)
    task_idx = task_idx.astype(jnp.int32).reshape(_NW, _NCHUNK, _CHUNK)
    theta_flat = theta.reshape(-1)
    beta_flat = beta.reshape(-1)
    out = _irt_sc(agent_idx, task_idx, theta_flat, beta_flat)
    return out.reshape(_BATCH)


# speedup vs baseline: 1.2148x; 1.0813x over previous
import functools

import jax
import jax.numpy as jnp
from jax import lax
from jax.experimental import pallas as pl
from jax.experimental.pallas import tpu as pltpu
from jax.experimental.pallas import tpu_sc as plsc

_BATCH = 16384

_info = plsc.get_sparse_core_info()
_NC = _info.num_cores
_NS = _info.num_subcores
_NW = _NC * _NS
_B_PER_W = _BATCH // _NW
_CHUNK = 128
_NCHUNK = _B_PER_W // _CHUNK
_LANES = 16


@functools.partial(
    pl.kernel,
    mesh=plsc.VectorSubcoreMesh(core_axis_name="c", subcore_axis_name="s"),
    out_type=jax.ShapeDtypeStruct((_BATCH,), jnp.float32),
    compiler_params=pltpu.CompilerParams(use_tc_tiling_on_sc=False),
    scratch_types=[
        pltpu.VMEM((_B_PER_W,), jnp.int32),
        pltpu.VMEM((_B_PER_W,), jnp.int32),
        pltpu.VMEM((_B_PER_W,), jnp.float32),
        pltpu.VMEM((_B_PER_W,), jnp.float32),
        pltpu.SemaphoreType.DMA,
        pltpu.SemaphoreType.DMA,
        pltpu.SemaphoreType.DMA,
    ],
)
def _irt_sc(aidx_hbm, tidx_hbm, th_hbm, be_hbm, out_hbm,
            aidx_v, tidx_v, th_v, be_v, sem_ai, sem_ti, sem_g):
    wid = lax.axis_index("s") * _NC + lax.axis_index("c")
    base = wid * _B_PER_W

    th_flat = th_hbm.at[0]
    be_flat = be_hbm.at[0]

    cp_a = pltpu.async_copy(aidx_hbm.at[pl.ds(base, _B_PER_W)], aidx_v, sem_ai)
    cp_t = pltpu.async_copy(tidx_hbm.at[pl.ds(base, _B_PER_W)], tidx_v, sem_ti)

    gathers = []
    cp_a.wait()
    for j in range(_NCHUNK):
        sl = pl.ds(j * _CHUNK, _CHUNK)
        gathers.append(
            pltpu.async_copy(th_flat.at[aidx_v.at[sl]], th_v.at[sl], sem_g))
    cp_t.wait()
    for j in range(_NCHUNK):
        sl = pl.ds(j * _CHUNK, _CHUNK)
        gathers.append(
            pltpu.async_copy(be_flat.at[tidx_v.at[sl]], be_v.at[sl], sem_g))
    for cp in gathers:
        cp.wait()

    for i in range(_B_PER_W // _LANES):
        sl = pl.ds(i * _LANES, _LANES)
        th_v[sl] = th_v[sl] - be_v[sl]

    pltpu.sync_copy(th_v, out_hbm.at[pl.ds(base, _B_PER_W)])


def kernel(agent_idx, task_idx, theta, beta):
    agent_idx = agent_idx.astype(jnp.int32)
    task_idx = task_idx.astype(jnp.int32)
    return _irt_sc(agent_idx, task_idx, theta.T, beta.T)
